# baseline (speedup 1.0000x reference)
import jax
import jax.numpy as jnp
from jax.experimental import pallas as pl
from jax.experimental.pallas import tpu as pltpu

QB = 16


def _knn_body(f0_ref, f1_ref, pos1_ref, out_pos_ref, out_w_ref):
    q = pl.program_id(1)
    f0 = f0_ref[0]
    f1 = f1_ref[0]
    n = f1.shape[0]

    diff = jnp.abs(f0[:, None, :] - f1[None, :, :])
    dist = jnp.sum(diff, axis=-1)

    d0 = jnp.min(dist, axis=-1, keepdims=True)
    jidx = jax.lax.broadcasted_iota(jnp.int32, dist.shape, 1)
    idx = jnp.min(jnp.where(dist == d0, jidx, n), axis=-1, keepdims=True)
    masked = jnp.where(jidx == idx, jnp.float32(jnp.inf), dist)
    d1 = jnp.min(masked, axis=-1, keepdims=True)

    inv_w = 1.0 / (d1 / d0 - 1.0)

    onehot = (jidx == idx).astype(jnp.float32)
    pos1 = pos1_ref[0]
    matched = jnp.sum(onehot[:, :, None] * pos1[None, :, :], axis=1)

    out_pos_ref[0, pl.ds(q * QB, QB), :] = matched
    out_w_ref[0, pl.ds(q * QB, QB), :] = inv_w


def kernel(feat0, feat1, pos1):
    B, N, F = feat0.shape
    grid = (B, N // QB)
    out_pos, out_w = pl.pallas_call(
        _knn_body,
        grid=grid,
        in_specs=[
            pl.BlockSpec((1, QB, F), lambda b, q: (b, q, 0)),
            pl.BlockSpec((1, N, F), lambda b, q: (b, 0, 0)),
            pl.BlockSpec((1, N, 2), lambda b, q: (b, 0, 0)),
        ],
        out_specs=[
            pl.BlockSpec((1, N, 2), lambda b, q: (b, 0, 0)),
            pl.BlockSpec((1, N, 1), lambda b, q: (b, 0, 0)),
        ],
        out_shape=[
            jax.ShapeDtypeStruct((B, N, 2), jnp.float32),
            jax.ShapeDtypeStruct((B, N, 1), jnp.float32),
        ],
        compiler_params=pltpu.CompilerParams(
            dimension_semantics=("parallel", "arbitrary"),
        ),
    )(feat0, feat1, pos1)
    return out_pos, out_w[:, :, 0]


# speedup vs baseline: 5.4605x; 5.4605x over previous
import jax
import jax.numpy as jnp
from jax.experimental import pallas as pl
from jax.experimental.pallas import tpu as pltpu

QB = 16


def _knn_body(f0_ref, f1t_ref, pos1t_ref, out_pos_ref, out_w_ref):
    q = pl.program_id(1)
    f0 = f0_ref[0]
    f1t = f1t_ref[0]
    F, n = f1t.shape

    acc = jnp.abs(f0[:, 0:1] - f1t[0:1, :])
    for f in range(1, F):
        acc = acc + jnp.abs(f0[:, f:f + 1] - f1t[f:f + 1, :])

    d0 = jnp.min(acc, axis=-1, keepdims=True)
    jidx = jax.lax.broadcasted_iota(jnp.int32, acc.shape, 1)
    idx = jnp.min(jnp.where(acc == d0, jidx, n), axis=-1, keepdims=True)
    onehot = (jidx == idx).astype(jnp.float32)
    masked = jnp.where(onehot > 0, jnp.float32(jnp.inf), acc)
    d1 = jnp.min(masked, axis=-1, keepdims=True)

    inv_w = 1.0 / (d1 / d0 - 1.0)

    px = jnp.sum(onehot * pos1t_ref[0, 0:1, :], axis=-1, keepdims=True)
    py = jnp.sum(onehot * pos1t_ref[0, 1:2, :], axis=-1, keepdims=True)
    matched = jnp.concatenate([px, py], axis=-1)

    out_pos_ref[0, pl.ds(q * QB, QB), :] = matched
    out_w_ref[0, pl.ds(q * QB, QB), :] = inv_w


def kernel(feat0, feat1, pos1):
    B, N, F = feat0.shape
    f1t = jnp.swapaxes(feat1, 1, 2)
    pos1t = jnp.swapaxes(pos1, 1, 2)
    grid = (B, N // QB)
    out_pos, out_w = pl.pallas_call(
        _knn_body,
        grid=grid,
        in_specs=[
            pl.BlockSpec((1, QB, F), lambda b, q: (b, q, 0)),
            pl.BlockSpec((1, F, N), lambda b, q: (b, 0, 0)),
            pl.BlockSpec((1, 2, N), lambda b, q: (b, 0, 0)),
        ],
        out_specs=[
            pl.BlockSpec((1, N, 2), lambda b, q: (b, 0, 0)),
            pl.BlockSpec((1, N, 1), lambda b, q: (b, 0, 0)),
        ],
        out_shape=[
            jax.ShapeDtypeStruct((B, N, 2), jnp.float32),
            jax.ShapeDtypeStruct((B, N, 1), jnp.float32),
        ],
        compiler_params=pltpu.CompilerParams(
            dimension_semantics=("parallel", "arbitrary"),
        ),
    )(feat0, f1t, pos1t)
    return out_pos, out_w[:, :, 0]
